# baseline (speedup 1.0000x reference)
import functools

import jax
import jax.numpy as jnp
from jax import lax
from jax.experimental import pallas as pl
from jax.experimental.pallas import tpu as pltpu
from jax.experimental.pallas import tpu_sc as plsc

_VOCAB = 1000000
_EMBED = 64
_CTX = 200
_BATCH = 4096
_SEQ = 200

_NC = 2
_NS = 16
_NW = _NC * _NS
_SPW = _BATCH // _NW
_G0 = 104
_G1 = _SEQ - _G0


def _emb_kernel(tok_hbm, e_hbm, p_hbm, out_hbm,
                idx0, idx1, row0, row1, gath, outb,
                sem_g0, sem_g1, sem_s0, sem_s1, sem_f0, sem_f1):
    idx = (idx0, idx1)
    row = (row0, row1)
    sem_g = (sem_g0, sem_g1)
    sem_s = (sem_s0, sem_s1)
    sem_f = (sem_f0, sem_f1)
    wid = lax.axis_index("s") * _NC + lax.axis_index("c")
    sbase = wid * _SPW

    def prefetch(s, b):
        tbase = (sbase + s) * _SEQ
        pltpu.sync_copy(tok_hbm.at[pl.ds(tbase, _SEQ)], idx[b])
        for m in range(13):
            off = 184 if m == 12 else m * 16
            sl = pl.ds(off, 16)
            row[b][sl] = idx[b][sl] >> 1
        pltpu.async_copy(
            e_hbm.at[row[b].at[pl.ds(0, _G0)]],
            gath.at[b, pl.ds(0, _G0)],
            sem_g[b],
        )
        pltpu.async_copy(
            e_hbm.at[row[b].at[pl.ds(_G0, _G1)]],
            gath.at[b, pl.ds(_G0, _G1)],
            sem_g[b],
        )
        pltpu.async_copy(p_hbm, outb.at[b], sem_f[b])

    def wait_gathers(b):
        pltpu.make_async_copy(
            e_hbm.at[pl.ds(0, _SEQ)], gath.at[b], sem_g[b]
        ).wait()
        pltpu.make_async_copy(p_hbm, outb.at[b], sem_f[b]).wait()

    def wait_store(b):
        pltpu.make_async_copy(
            outb.at[b], out_hbm.at[0], sem_s[b]
        ).wait()

    def add_pos(b):
        gb = gath.at[b]
        ob = outb.at[b]

        def lanes(base, tvec, lo):
            ovec = (tvec & 1) * _EMBED
            for l in range(lo, 16):
                o = ovec[l]
                i = base + l
                for j in range(_EMBED // 16):
                    v = gb[i, pl.ds(o + j * 16, 16)]
                    plsc.addupdate(ob.at[i, pl.ds(j * 16, 16)], v)

        def group(g, c):
            base = g * 16
            lanes(base, idx[b][pl.ds(base, 16)], 0)
            return c

        lax.fori_loop(0, _SEQ // 16, group, 0)
        lanes(184, idx[b][pl.ds(184, 16)], 8)

    prefetch(0, 0)

    def outer(i, carry):
        for b in range(2):
            k = 2 * i + b
            bn = b ^ 1

            @pl.when(k >= 1)
            def _():
                wait_store(bn)

            @pl.when(k + 1 < _SPW)
            def _():
                prefetch(k + 1, bn)

            wait_gathers(b)
            add_pos(b)
            pltpu.async_copy(outb.at[b], out_hbm.at[sbase + k], sem_s[b])
        return carry

    lax.fori_loop(0, _SPW // 2, outer, 0)
    wait_store(1)


def kernel(token_batch, E, P):
    tok = token_batch.reshape(-1).astype(jnp.int32)
    e2 = E.reshape(_VOCAB // 2, 2 * _EMBED)
    mesh = plsc.VectorSubcoreMesh(core_axis_name="c", subcore_axis_name="s")
    run = functools.partial(
        pl.kernel,
        mesh=mesh,
        compiler_params=pltpu.CompilerParams(use_tc_tiling_on_sc=True),
        out_type=jax.ShapeDtypeStruct((_BATCH, _SEQ, _EMBED), jnp.float32),
        scratch_types=[
            pltpu.VMEM((_SEQ,), jnp.int32),
            pltpu.VMEM((_SEQ,), jnp.int32),
            pltpu.VMEM((_SEQ,), jnp.int32),
            pltpu.VMEM((_SEQ,), jnp.int32),
            pltpu.VMEM((2, _SEQ, 2 * _EMBED), jnp.float32),
            pltpu.VMEM((2, _SEQ, _EMBED), jnp.float32),
            pltpu.SemaphoreType.DMA,
            pltpu.SemaphoreType.DMA,
            pltpu.SemaphoreType.DMA,
            pltpu.SemaphoreType.DMA,
            pltpu.SemaphoreType.DMA,
            pltpu.SemaphoreType.DMA,
        ],
    )(_emb_kernel)
    return run(tok, e2, P)
